# baseline (speedup 1.0000x reference)
import jax
import jax.numpy as jnp
from jax import lax
from jax.experimental import pallas as pl
from jax.experimental.pallas import tpu as pltpu
from jax.experimental.pallas import tpu_sc as plsc

_SMOOTHING = 0.1
_L = 16


def kernel(target, pred):
    b, k = pred.shape
    low = _SMOOTHING / k
    hi = low + (1.0 - _SMOOTHING)

    mesh = plsc.VectorSubcoreMesh(core_axis_name="c", subcore_axis_name="s")
    nw = mesh.num_cores * mesh.num_subcores
    rpw = b // nw

    def body(target_hbm, out_hbm, buf, tgt_v, pidx, vals, sem):
        wid = lax.axis_index("s") * mesh.num_cores + lax.axis_index("c")
        base = wid * rpw
        pltpu.sync_copy(target_hbm.at[pl.ds(base, rpw)], tgt_v)

        low_v = jnp.full((_L,), low, jnp.float32)
        hi_v = jnp.full((_L,), hi, jnp.float32)
        lane_ids = jnp.arange(_L, dtype=jnp.int32)

        def fill(i, carry):
            buf[pl.ds(i * _L, _L)] = low_v
            return carry

        lax.fori_loop(0, k // _L, fill, 0)

        for ci in range(rpw // _L):
            tv = tgt_v[pl.ds(ci * _L, _L)]
            rows = base + ci * _L + lane_ids
            pidx[pl.ds(ci * _L, _L)] = rows * k + tv
            vals[pl.ds(ci * _L, _L)] = hi_v

        for i in range(rpw):
            pltpu.make_async_copy(buf, out_hbm.at[pl.ds((base + i) * k, k)], sem).start()
        for i in range(rpw):
            pltpu.make_async_copy(buf, out_hbm.at[pl.ds((base + i) * k, k)], sem).wait()

        pltpu.sync_copy(vals, out_hbm.at[pidx])

    f = pl.kernel(
        body,
        out_type=jax.ShapeDtypeStruct((b * k,), jnp.float32),
        mesh=mesh,
        scratch_types=[
            pltpu.VMEM((k,), jnp.float32),
            pltpu.VMEM((rpw,), jnp.int32),
            pltpu.VMEM((rpw,), jnp.int32),
            pltpu.VMEM((rpw,), jnp.float32),
            pltpu.SemaphoreType.DMA,
        ],
        compiler_params=pltpu.CompilerParams(needs_layout_passes=False),
    )
    return f(target).reshape(b, k)
